# Initial kernel scaffold; baseline (speedup 1.0000x reference)
import jax
import jax.numpy as jnp
from jax.experimental import pallas as pl


def kernel(x, edge_index, edge_values, W, b):
    raise NotImplementedError("write your pallas kernel here")


# speedup vs baseline: 5.2384x; 5.2384x over previous
import functools

import jax
import jax.numpy as jnp
from jax import lax
from jax.experimental import pallas as pl
from jax.experimental.pallas import tpu as pltpu
from jax.experimental.pallas import tpu_sc as plsc

N_NODES = 10000
N_EDGES = 320000
D_IN = 128
D_OUT = 128

NC = 2
NS = 16
NW = NC * NS
LANES = 16

K = 128
EPW = 10240
NCHUNK = EPW // K
E_PAD = EPW * NW
ROWS_PER_SUB = 624
ROWS_TAIL = N_NODES - ROWS_PER_SUB * NS


def _sc_spmm(support, cols, rows, vals, zeros):
    mesh = plsc.VectorSubcoreMesh(
        core_axis_name="c", subcore_axis_name="s", num_cores=NC, num_subcores=NS
    )

    @functools.partial(
        pl.kernel,
        mesh=mesh,
        out_type=jax.ShapeDtypeStruct((NC, N_NODES, D_OUT), jnp.float32),
        scratch_types=[
            pltpu.VMEM_SHARED((N_NODES, D_OUT), jnp.float32),
            pltpu.VMEM((K,), jnp.int32),
            pltpu.VMEM((K,), jnp.int32),
            pltpu.VMEM((K,), jnp.float32),
            pltpu.VMEM((K, D_OUT), jnp.float32),
            pltpu.SemaphoreType.DMA,
        ],
    )
    def spmm(support_hbm, cols_hbm, rows_hbm, vals_hbm, zeros_hbm, out_hbm,
             acc, cidx, ridx, ev, gath, sem):
        c = lax.axis_index("c")
        s = lax.axis_index("s")
        wid = s * NC + c

        rbase = s * ROWS_PER_SUB
        pltpu.sync_copy(
            zeros_hbm.at[pl.ds(rbase, ROWS_PER_SUB), :],
            acc.at[pl.ds(rbase, ROWS_PER_SUB), :],
        )

        @pl.when(s == NS - 1)
        def _():
            tb = NS * ROWS_PER_SUB
            pltpu.sync_copy(
                zeros_hbm.at[pl.ds(tb, ROWS_TAIL), :],
                acc.at[pl.ds(tb, ROWS_TAIL), :],
            )

        plsc.subcore_barrier()

        ebase = wid * EPW

        def chunk_body(k, carry):
            off = ebase + k * K
            pltpu.sync_copy(cols_hbm.at[pl.ds(off, K)], cidx)
            pltpu.sync_copy(rows_hbm.at[pl.ds(off, K)], ridx)
            pltpu.sync_copy(vals_hbm.at[pl.ds(off, K)], ev)
            pltpu.async_copy(support_hbm.at[cidx], gath, sem).wait()

            def group_body(g, carry2):
                v16 = ev[pl.ds(g * LANES, LANES)]
                for l in range(LANES):
                    val = jnp.broadcast_to(v16[l], (LANES,))
                    e = g * LANES + l
                    for j in range(D_OUT // LANES):
                        sl = pl.ds(j * LANES, LANES)
                        gath[e, sl] = gath[e, sl] * val
                return carry2

            lax.fori_loop(0, K // LANES, group_body, 0)
            pltpu.sync_copy(gath, acc.at[ridx], add=True)
            return carry

        lax.fori_loop(0, NCHUNK, chunk_body, 0)

        plsc.subcore_barrier()
        pltpu.sync_copy(
            acc.at[pl.ds(rbase, ROWS_PER_SUB), :],
            out_hbm.at[c, pl.ds(rbase, ROWS_PER_SUB), :],
        )

        @pl.when(s == NS - 1)
        def _():
            tb = NS * ROWS_PER_SUB
            pltpu.sync_copy(
                acc.at[pl.ds(tb, ROWS_TAIL), :],
                out_hbm.at[c, pl.ds(tb, ROWS_TAIL), :],
            )

    return spmm(support, cols, rows, vals, zeros)


def _matmul(x, W):
    def body(x_ref, w_ref, o_ref):
        o_ref[...] = jnp.dot(x_ref[...], w_ref[...],
                             preferred_element_type=jnp.float32)

    return pl.pallas_call(
        body,
        grid=(10,),
        in_specs=[
            pl.BlockSpec((N_NODES // 10, D_IN), lambda i: (i, 0)),
            pl.BlockSpec((D_IN, D_OUT), lambda i: (0, 0)),
        ],
        out_specs=pl.BlockSpec((N_NODES // 10, D_OUT), lambda i: (i, 0)),
        out_shape=jax.ShapeDtypeStruct((N_NODES, D_OUT), jnp.float32),
    )(x, W)


def _merge(partials, b):
    def body(p_ref, b_ref, o_ref):
        o_ref[...] = p_ref[0] + p_ref[1] + b_ref[...]

    return pl.pallas_call(
        body,
        grid=(10,),
        in_specs=[
            pl.BlockSpec((NC, N_NODES // 10, D_OUT), lambda i: (0, i, 0)),
            pl.BlockSpec((1, D_OUT), lambda i: (0, 0)),
        ],
        out_specs=pl.BlockSpec((N_NODES // 10, D_OUT), lambda i: (i, 0)),
        out_shape=jax.ShapeDtypeStruct((N_NODES, D_OUT), jnp.float32),
    )(partials, b.reshape(1, D_OUT))


def kernel(x, edge_index, edge_values, W, b):
    rows = edge_index[0].astype(jnp.int32)
    cols = edge_index[1].astype(jnp.int32)

    pad = E_PAD - N_EDGES
    padidx = jnp.arange(pad, dtype=jnp.int32) % N_NODES
    cols_p = jnp.concatenate([cols, padidx])
    rows_p = jnp.concatenate([rows, padidx])
    vals_p = jnp.concatenate([edge_values, jnp.zeros((pad,), jnp.float32)])
    zeros = jnp.zeros((N_NODES, D_OUT), jnp.float32)

    support = _matmul(x, W)
    partials = _sc_spmm(support, cols_p, rows_p, vals_p, zeros)
    return _merge(partials, b)


# speedup vs baseline: 9.5807x; 1.8289x over previous
import functools

import jax
import jax.numpy as jnp
from jax import lax
from jax.experimental import pallas as pl
from jax.experimental.pallas import tpu as pltpu
from jax.experimental.pallas import tpu_sc as plsc

N_NODES = 10000
N_EDGES = 320000
D_IN = 128
D_OUT = 128

NC = 2
NS = 16
NW = NC * NS
LANES = 16

K = 128
EPW = 10240
NCHUNK = EPW // K
E_PAD = EPW * NW
ROWS_PER_SUB = 624
ROWS_TAIL = N_NODES - ROWS_PER_SUB * NS


def _sc_spmm(support, cols, vals, rows):
    mesh = plsc.VectorSubcoreMesh(
        core_axis_name="c", subcore_axis_name="s", num_cores=NC, num_subcores=NS
    )

    @functools.partial(
        pl.kernel,
        mesh=mesh,
        out_type=jax.ShapeDtypeStruct((NC, N_NODES, D_OUT), jnp.float32),
        scratch_types=[
            pltpu.VMEM_SHARED((N_NODES, D_OUT), jnp.float32),
            pltpu.VMEM((K,), jnp.int32),
            pltpu.VMEM((K,), jnp.int32),
            pltpu.VMEM((K,), jnp.float32),
            pltpu.VMEM((K,), jnp.float32),
            pltpu.VMEM((K,), jnp.int32),
            pltpu.VMEM((K,), jnp.int32),
            pltpu.VMEM((K, D_OUT), jnp.float32),
            pltpu.VMEM((K, D_OUT), jnp.float32),
            pltpu.SemaphoreType.DMA,
            pltpu.SemaphoreType.DMA,
            pltpu.SemaphoreType.DMA,
            pltpu.SemaphoreType.DMA,
            pltpu.SemaphoreType.DMA,
            pltpu.SemaphoreType.DMA,
            pltpu.SemaphoreType.DMA,
            pltpu.SemaphoreType.DMA,
        ],
    )
    def spmm(support_hbm, cols_hbm, vals_hbm, rows_hbm, out_hbm,
             acc, pb0, pb1, vb0, vb1, rx0, rx1, gath0, gath1,
             gsem0, gsem1, ssem0, ssem1, isem0, isem1, rsem0, rsem1):
        c = lax.axis_index("c")
        s = lax.axis_index("s")
        wid = s * NC + c

        zvec = jnp.zeros((LANES,), jnp.float32)

        def zrow(r, carry):
            for j in range(D_OUT // LANES):
                gath0[r, pl.ds(j * LANES, LANES)] = zvec
            return carry

        lax.fori_loop(0, K, zrow, 0)
        rbase = s * ROWS_PER_SUB
        for blk in range(ROWS_PER_SUB // K):
            pltpu.sync_copy(gath0, acc.at[pl.ds(rbase + blk * K, K), :])
        rem = ROWS_PER_SUB % K
        if rem:
            pltpu.sync_copy(
                gath0.at[pl.ds(0, rem), :],
                acc.at[pl.ds(rbase + (ROWS_PER_SUB // K) * K, rem), :],
            )

        @pl.when(s == NS - 1)
        def _():
            tb = NS * ROWS_PER_SUB
            pltpu.sync_copy(
                gath0.at[pl.ds(0, ROWS_TAIL), :],
                acc.at[pl.ds(tb, ROWS_TAIL), :],
            )

        plsc.subcore_barrier()

        set_a = (pb0, vb0, rx0, gath0, gsem0, ssem0, isem0, rsem0)
        set_b = (pb1, vb1, rx1, gath1, gsem1, ssem1, isem1, rsem1)
        pbase = wid * NCHUNK

        def pb_copy(k, st):
            return pltpu.make_async_copy(
                cols_hbm.at[pl.ds((pbase + k) * K, K)], st[0], st[6])

        def vb_copy(k, st):
            return pltpu.make_async_copy(
                vals_hbm.at[pl.ds((pbase + k) * K, K)], st[1], st[6])

        def rx_copy(k, st):
            return pltpu.make_async_copy(
                rows_hbm.at[pl.ds((pbase + k) * K, K)], st[2], st[7])

        def gather_copy(st):
            return pltpu.make_async_copy(support_hbm.at[st[0]], st[3], st[4])

        def start_scatter(st):
            pltpu.async_copy(st[3], acc.at[st[2]], st[5], add=True)

        def wait_scatter(st):
            pltpu.make_async_copy(st[3], acc.at[st[2]], st[5]).wait()

        NG = K // LANES
        SPLIT = 5

        def scale_part(gath, vb, g_lo, g_hi):
            def group_body(g, carry2):
                v16 = vb[pl.ds(g * LANES, LANES)]
                for l in range(LANES):
                    val = jnp.broadcast_to(v16[l], (LANES,))
                    e = g * LANES + l
                    for j in range(D_OUT // LANES):
                        sl = pl.ds(j * LANES, LANES)
                        gath[e, sl] = gath[e, sl] * val
                return carry2

            lax.fori_loop(g_lo, g_hi, group_body, 0)

        def step(k, cur, nxt, wait_nxt, prefetch):
            if prefetch:
                pb_copy(k + 1, nxt).start()
                vb_copy(k + 1, nxt).start()
            gather_copy(cur).wait()
            scale_part(cur[3], cur[1], 0, SPLIT)
            if wait_nxt:
                wait_scatter(nxt)
            if prefetch:
                rx_copy(k + 1, nxt).start()
                pb_copy(k + 1, nxt).wait()
                vb_copy(k + 1, nxt).wait()
                gather_copy(nxt).start()
            scale_part(cur[3], cur[1], SPLIT, NG)
            rx_copy(k, cur).wait()
            start_scatter(cur)

        pb_copy(0, set_a).start()
        vb_copy(0, set_a).start()
        rx_copy(0, set_a).start()
        pb_copy(0, set_a).wait()
        vb_copy(0, set_a).wait()
        gather_copy(set_a).start()
        step(0, set_a, set_b, wait_nxt=False, prefetch=True)

        def pair_body(t, carry):
            step(2 * t + 1, set_b, set_a, wait_nxt=True, prefetch=True)
            step(2 * t + 2, set_a, set_b, wait_nxt=True, prefetch=True)
            return carry

        lax.fori_loop(0, (NCHUNK - 2) // 2, pair_body, 0)

        step(NCHUNK - 1, set_b, set_a, wait_nxt=True, prefetch=False)
        wait_scatter(set_b)

        plsc.subcore_barrier()
        pltpu.sync_copy(
            acc.at[pl.ds(rbase, ROWS_PER_SUB), :],
            out_hbm.at[c, pl.ds(rbase, ROWS_PER_SUB), :],
        )

        @pl.when(s == NS - 1)
        def _():
            tb = NS * ROWS_PER_SUB
            pltpu.sync_copy(
                acc.at[pl.ds(tb, ROWS_TAIL), :],
                out_hbm.at[c, pl.ds(tb, ROWS_TAIL), :],
            )

    return spmm(support, cols, vals, rows)


def _matmul(x, W):
    def body(x_ref, w_ref, o_ref):
        o_ref[...] = jnp.dot(x_ref[...], w_ref[...],
                             preferred_element_type=jnp.float32)

    return pl.pallas_call(
        body,
        grid=(10,),
        in_specs=[
            pl.BlockSpec((N_NODES // 10, D_IN), lambda i: (i, 0)),
            pl.BlockSpec((D_IN, D_OUT), lambda i: (0, 0)),
        ],
        out_specs=pl.BlockSpec((N_NODES // 10, D_OUT), lambda i: (i, 0)),
        out_shape=jax.ShapeDtypeStruct((N_NODES, D_OUT), jnp.float32),
    )(x, W)


def _merge(partials, b):
    def body(p_ref, b_ref, o_ref):
        o_ref[...] = p_ref[0] + p_ref[1] + b_ref[...]

    return pl.pallas_call(
        body,
        grid=(10,),
        in_specs=[
            pl.BlockSpec((NC, N_NODES // 10, D_OUT), lambda i: (0, i, 0)),
            pl.BlockSpec((1, D_OUT), lambda i: (0, 0)),
        ],
        out_specs=pl.BlockSpec((N_NODES // 10, D_OUT), lambda i: (i, 0)),
        out_shape=jax.ShapeDtypeStruct((N_NODES, D_OUT), jnp.float32),
    )(partials, b.reshape(1, D_OUT))


def kernel(x, edge_index, edge_values, W, b):
    rows = edge_index[0].astype(jnp.int32)
    cols = edge_index[1].astype(jnp.int32)

    pad = E_PAD - N_EDGES
    padidx = jnp.arange(pad, dtype=jnp.int32) % N_NODES
    cols_p = jnp.concatenate([cols, padidx])
    rows_p = jnp.concatenate([rows, padidx])
    vals_p = jnp.concatenate([edge_values, jnp.zeros((pad,), jnp.float32)])

    support = _matmul(x, W)
    partials = _sc_spmm(support, cols_p, vals_p, rows_p)
    return _merge(partials, b)


# speedup vs baseline: 11.2404x; 1.1732x over previous
import functools

import jax
import jax.numpy as jnp
from jax import lax
from jax.experimental import pallas as pl
from jax.experimental.pallas import tpu as pltpu
from jax.experimental.pallas import tpu_sc as plsc

N_NODES = 10000
N_EDGES = 320000
D_IN = 128
D_OUT = 128

NC = 2
NS = 16
NW = NC * NS
LANES = 16

K = 64
EPW = 10240
NCHUNK = EPW // K
E_PAD = EPW * NW
NSETS = 4
ROWS_PER_SUB = 624
ROWS_TAIL = N_NODES - ROWS_PER_SUB * NS


def _sc_spmm(support, cols, vals, rows):
    mesh = plsc.VectorSubcoreMesh(
        core_axis_name="c", subcore_axis_name="s", num_cores=NC, num_subcores=NS
    )

    scratch = [pltpu.VMEM_SHARED((N_NODES, D_OUT), jnp.float32)]
    for _ in range(NSETS):
        scratch += [
            pltpu.VMEM((K,), jnp.int32),
            pltpu.VMEM((K,), jnp.float32),
            pltpu.VMEM((K,), jnp.int32),
            pltpu.VMEM((K, D_OUT), jnp.float32),
            pltpu.SemaphoreType.DMA,
            pltpu.SemaphoreType.DMA,
            pltpu.SemaphoreType.DMA,
        ]

    @functools.partial(
        pl.kernel,
        mesh=mesh,
        out_type=jax.ShapeDtypeStruct((NC, N_NODES, D_OUT), jnp.float32),
        scratch_types=scratch,
    )
    def spmm(support_hbm, cols_hbm, vals_hbm, rows_hbm, out_hbm, acc, *bufs):
        c = lax.axis_index("c")
        s = lax.axis_index("s")
        wid = s * NC + c
        sets = tuple(bufs[i * 7:(i + 1) * 7] for i in range(NSETS))
        zbuf = sets[0][3]

        zvec = jnp.zeros((LANES,), jnp.float32)

        def zrow(r, carry):
            for j in range(D_OUT // LANES):
                zbuf[r, pl.ds(j * LANES, LANES)] = zvec
            return carry

        lax.fori_loop(0, K, zrow, 0)
        rbase = s * ROWS_PER_SUB
        for blk in range(ROWS_PER_SUB // K):
            pltpu.sync_copy(zbuf, acc.at[pl.ds(rbase + blk * K, K), :])
        rem = ROWS_PER_SUB % K
        if rem:
            pltpu.sync_copy(
                zbuf.at[pl.ds(0, rem), :],
                acc.at[pl.ds(rbase + (ROWS_PER_SUB // K) * K, rem), :],
            )

        @pl.when(s == NS - 1)
        def _():
            tb = NS * ROWS_PER_SUB
            pltpu.sync_copy(
                zbuf.at[pl.ds(0, ROWS_TAIL), :],
                acc.at[pl.ds(tb, ROWS_TAIL), :],
            )

        plsc.subcore_barrier()

        pbase = wid * NCHUNK

        def idx_copies(k, st):
            sl = pl.ds((pbase + k) * K, K)
            return (pltpu.make_async_copy(cols_hbm.at[sl], st[0], st[4]),
                    pltpu.make_async_copy(vals_hbm.at[sl], st[1], st[4]),
                    pltpu.make_async_copy(rows_hbm.at[sl], st[2], st[4]))

        def start_idx(k, st):
            for d in idx_copies(k, st):
                d.start()

        def wait_idx(k, st):
            for d in idx_copies(k, st):
                d.wait()

        def gather_copy(st):
            return pltpu.make_async_copy(support_hbm.at[st[0]], st[3], st[5])

        def start_scatter(st):
            pltpu.async_copy(st[3], acc.at[st[2]], st[6], add=True)

        def wait_scatter(st):
            pltpu.make_async_copy(st[3], acc.at[st[2]], st[6]).wait()

        def scale(st):
            gath, vb = st[3], st[1]

            def group_body(g, carry2):
                v16 = vb[pl.ds(g * LANES, LANES)]
                for l in range(LANES):
                    val = jnp.broadcast_to(v16[l], (LANES,))
                    e = g * LANES + l
                    for j in range(D_OUT // LANES):
                        sl = pl.ds(j * LANES, LANES)
                        gath[e, sl] = gath[e, sl] * val
                return carry2

            lax.fori_loop(0, K // LANES, group_body, 0)

        def step(k, i, drain, prefetch_idx, prefetch_gather):
            cur = sets[i]
            if drain:
                wait_scatter(sets[(i + 2) % NSETS])
            if prefetch_idx:
                start_idx(k + 2, sets[(i + 2) % NSETS])
            if prefetch_gather:
                nxt = sets[(i + 1) % NSETS]
                wait_idx(k + 1, nxt)
                gather_copy(nxt).start()
            gather_copy(cur).wait()
            scale(cur)
            start_scatter(cur)

        start_idx(0, sets[0])
        start_idx(1, sets[1])
        wait_idx(0, sets[0])
        gather_copy(sets[0]).start()

        step(0, 0, drain=False, prefetch_idx=True, prefetch_gather=True)
        step(1, 1, drain=False, prefetch_idx=True, prefetch_gather=True)

        def quad_body(t, carry):
            k = 4 * t + 2
            for i in range(NSETS):
                step(k + i, (2 + i) % NSETS, drain=True,
                     prefetch_idx=True, prefetch_gather=True)
            return carry

        lax.fori_loop(0, (NCHUNK - 4) // 4, quad_body, 0)

        step(NCHUNK - 2, (NCHUNK - 2) % NSETS, drain=True,
             prefetch_idx=False, prefetch_gather=True)
        step(NCHUNK - 1, (NCHUNK - 1) % NSETS, drain=True,
             prefetch_idx=False, prefetch_gather=False)
        wait_scatter(sets[(NCHUNK - 2) % NSETS])
        wait_scatter(sets[(NCHUNK - 1) % NSETS])

        plsc.subcore_barrier()
        pltpu.sync_copy(
            acc.at[pl.ds(rbase, ROWS_PER_SUB), :],
            out_hbm.at[c, pl.ds(rbase, ROWS_PER_SUB), :],
        )

        @pl.when(s == NS - 1)
        def _():
            tb = NS * ROWS_PER_SUB
            pltpu.sync_copy(
                acc.at[pl.ds(tb, ROWS_TAIL), :],
                out_hbm.at[c, pl.ds(tb, ROWS_TAIL), :],
            )

    return spmm(support, cols, vals, rows)


def _matmul(x, W):
    def body(x_ref, w_ref, o_ref):
        o_ref[...] = jnp.dot(x_ref[...], w_ref[...],
                             preferred_element_type=jnp.float32)

    return pl.pallas_call(
        body,
        grid=(10,),
        in_specs=[
            pl.BlockSpec((N_NODES // 10, D_IN), lambda i: (i, 0)),
            pl.BlockSpec((D_IN, D_OUT), lambda i: (0, 0)),
        ],
        out_specs=pl.BlockSpec((N_NODES // 10, D_OUT), lambda i: (i, 0)),
        out_shape=jax.ShapeDtypeStruct((N_NODES, D_OUT), jnp.float32),
    )(x, W)


def _merge(partials, b):
    def body(p_ref, b_ref, o_ref):
        o_ref[...] = p_ref[0] + p_ref[1] + b_ref[...]

    return pl.pallas_call(
        body,
        grid=(10,),
        in_specs=[
            pl.BlockSpec((NC, N_NODES // 10, D_OUT), lambda i: (0, i, 0)),
            pl.BlockSpec((1, D_OUT), lambda i: (0, 0)),
        ],
        out_specs=pl.BlockSpec((N_NODES // 10, D_OUT), lambda i: (i, 0)),
        out_shape=jax.ShapeDtypeStruct((N_NODES, D_OUT), jnp.float32),
    )(partials, b.reshape(1, D_OUT))


def kernel(x, edge_index, edge_values, W, b):
    rows = edge_index[0].astype(jnp.int32)
    cols = edge_index[1].astype(jnp.int32)

    pad = E_PAD - N_EDGES
    padidx = jnp.arange(pad, dtype=jnp.int32) % N_NODES
    cols_p = jnp.concatenate([cols, padidx])
    rows_p = jnp.concatenate([rows, padidx])
    vals_p = jnp.concatenate([edge_values, jnp.zeros((pad,), jnp.float32)])

    support = _matmul(x, W)
    partials = _sc_spmm(support, cols_p, vals_p, rows_p)
    return _merge(partials, b)
